# Initial kernel scaffold; baseline (speedup 1.0000x reference)
import functools

import jax
import jax.numpy as jnp
from jax import lax
from jax.experimental import pallas as pl
from jax.experimental.pallas import tpu as pltpu
from jax.experimental.pallas import tpu_sc as plsc

N = 10000
E = 320000
D = 128
ALPHA = 0.1
K_PROP = 10

NC = 2
NS = 16
NW = NC * NS

NPAD = 10240
DEG_PER_W = NPAD // NS

CHUNK = 100
NCHUNKS = (E // NW) // CHUNK

DCHUNK = 80
DNCHUNKS = (E // NS) // DCHUNK

ROWS_PER_W = N // NS
ZROWS = 125

_mesh = plsc.VectorSubcoreMesh(core_axis_name="c", subcore_axis_name="s")


def _fill(ref, nwords, value):
    v = jnp.full((16,), value, ref.dtype)
    for i in range(nwords // 16):
        ref[pl.ds(i * 16, 16)] = v


@functools.partial(
    pl.kernel,
    out_type=jax.ShapeDtypeStruct((NC, NPAD), jnp.float32),
    mesh=_mesh,
    scratch_types=[
        pltpu.VMEM((DNCHUNKS, DCHUNK), jnp.int32),
        pltpu.VMEM((DCHUNK,), jnp.float32),
        pltpu.VMEM((DEG_PER_W,), jnp.float32),
        pltpu.VMEM_SHARED((NPAD,), jnp.float32),
    ],
)
def _deg_kernel(ed_hbm, out_hbm, idx_v, ones_v, zeros_v, acc_sh):
    c = lax.axis_index("c")
    s = lax.axis_index("s")
    _fill(ones_v, DCHUNK, 1.0)
    _fill(zeros_v, DEG_PER_W, 0.0)
    pltpu.sync_copy(zeros_v, acc_sh.at[pl.ds(s * DEG_PER_W, DEG_PER_W)])
    plsc.subcore_barrier()
    pltpu.sync_copy(ed_hbm.at[c, s], idx_v)

    def chunk(j, carry):
        pltpu.sync_copy(ones_v, acc_sh.at[idx_v.at[j]], add=True)
        return carry

    lax.fori_loop(0, DNCHUNKS, chunk, 0, unroll=False)
    plsc.subcore_barrier()
    sl = pl.ds(s * DEG_PER_W, DEG_PER_W)
    pltpu.sync_copy(acc_sh.at[sl], out_hbm.at[c, sl])


@functools.partial(
    pl.kernel,
    out_type=jax.ShapeDtypeStruct((NC, N, D), jnp.float32),
    mesh=_mesh,
    scratch_types=[
        pltpu.VMEM((NCHUNKS, CHUNK), jnp.int32),
        pltpu.VMEM((NCHUNKS, CHUNK), jnp.int32),
        pltpu.VMEM((CHUNK, D), jnp.float32),
        pltpu.VMEM((ZROWS, D), jnp.float32),
        pltpu.VMEM_SHARED((N, D), jnp.float32),
        pltpu.SemaphoreType.DMA,
    ],
)
def _spmm_kernel(g_hbm, src_hbm, dst_hbm, out_hbm,
                 src_v, dst_v, rows_v, zbuf_v, acc_sh, gsem):
    c = lax.axis_index("c")
    s = lax.axis_index("s")
    wid = s * NC + c

    def zrow(i, carry):
        for l in range(D // 16):
            zbuf_v[i, pl.ds(l * 16, 16)] = jnp.zeros((16,), jnp.float32)
        return carry

    lax.fori_loop(0, ZROWS, zrow, 0, unroll=False)
    base = s * ROWS_PER_W
    for i in range(ROWS_PER_W // ZROWS):
        pltpu.sync_copy(zbuf_v, acc_sh.at[pl.ds(base + i * ZROWS, ZROWS)])
    plsc.subcore_barrier()

    pltpu.sync_copy(src_hbm.at[wid], src_v)
    pltpu.sync_copy(dst_hbm.at[wid], dst_v)

    def chunk(j, carry):
        pltpu.async_copy(g_hbm.at[src_v.at[j]], rows_v, gsem).wait()
        pltpu.sync_copy(rows_v, acc_sh.at[dst_v.at[j]], add=True)
        return carry

    lax.fori_loop(0, NCHUNKS, chunk, 0, unroll=False)
    plsc.subcore_barrier()

    for i in range(ROWS_PER_W // ZROWS):
        sl = pl.ds(base + i * ZROWS, ZROWS)
        pltpu.sync_copy(acc_sh.at[sl], out_hbm.at[c, sl])


def _norm_body(deg_ref, out_ref):
    d = deg_ref[...]
    out_ref[...] = jnp.where(d > 0, lax.rsqrt(jnp.maximum(d, 1e-12)), 0.0)


def _norms_call(deg):
    return pl.pallas_call(
        _norm_body,
        out_shape=jax.ShapeDtypeStruct((NC, NPAD), jnp.float32),
    )(deg)


def _mm_body(x_ref, w_ref, ns_ref, o_ref):
    xw = jnp.dot(x_ref[...], w_ref[...], preferred_element_type=jnp.float32)
    o_ref[...] = xw * ns_ref[...]


def _mm_call(x, w, ns_col):
    grid = 10
    blk = N // grid
    return pl.pallas_call(
        _mm_body,
        grid=(grid,),
        in_specs=[
            pl.BlockSpec((blk, D), lambda i: (i, 0)),
            pl.BlockSpec((D, D), lambda i: (0, 0)),
            pl.BlockSpec((blk, 1), lambda i: (i, 0)),
        ],
        out_specs=pl.BlockSpec((blk, D), lambda i: (i, 0)),
        out_shape=jax.ShapeDtypeStruct((N, D), jnp.float32),
    )(x, w, ns_col)


def _gcn_body(p_ref, b_ref, nd_ref, ns_ref, h_ref, g_ref):
    t = (p_ref[0] + p_ref[1]) * nd_ref[...]
    h = jnp.maximum(t + b_ref[...], 0.0)
    h_ref[...] = h
    g_ref[...] = h * ns_ref[...]


def _appnp_body(p_ref, h0_ref, nd_ref, ns_ref, h_ref, g_ref):
    t = (p_ref[0] + p_ref[1]) * nd_ref[...]
    h = (1.0 - ALPHA) * t + ALPHA * h0_ref[...]
    h_ref[...] = h
    g_ref[...] = h * ns_ref[...]


def _combine_call(body, p, extra, extra_is_full, nd_col, ns_col):
    grid = 10
    blk = N // grid
    col = pl.BlockSpec((blk, 1), lambda i: (i, 0))
    mat = pl.BlockSpec((blk, D), lambda i: (i, 0))
    extra_spec = mat if extra_is_full else pl.BlockSpec((1, D), lambda i: (0, 0))
    return pl.pallas_call(
        body,
        grid=(grid,),
        in_specs=[
            pl.BlockSpec((NC, blk, D), lambda i: (0, i, 0)),
            extra_spec, col, col,
        ],
        out_specs=[mat, mat],
        out_shape=[jax.ShapeDtypeStruct((N, D), jnp.float32)] * 2,
    )(p, extra, nd_col, ns_col)


@jax.jit
def kernel(features, edge_index, W, b):
    src = edge_index[0].astype(jnp.int32)
    dst = edge_index[1].astype(jnp.int32)
    ed = jnp.stack([src, dst]).reshape(2, NS, DNCHUNKS, DCHUNK)
    srcr = src.reshape(NW, NCHUNKS, CHUNK)
    dstr = dst.reshape(NW, NCHUNKS, CHUNK)

    deg = _deg_kernel(ed)
    norms = _norms_call(deg)
    ns_col = norms[0, :N].reshape(N, 1)
    nd_col = norms[1, :N].reshape(N, 1)

    g = _mm_call(features, W, ns_col)
    p = _spmm_kernel(g, srcr, dstr)
    h, g = _combine_call(_gcn_body, p, b.reshape(1, D), False, nd_col, ns_col)
    h0 = h
    for _ in range(K_PROP):
        p = _spmm_kernel(g, srcr, dstr)
        h, g = _combine_call(_appnp_body, p, h0, True, nd_col, ns_col)
    return h


# speedup vs baseline: 7.0665x; 7.0665x over previous
import functools

import jax
import jax.numpy as jnp
from jax import lax
from jax.experimental import pallas as pl
from jax.experimental.pallas import tpu as pltpu
from jax.experimental.pallas import tpu_sc as plsc

N = 10000
E = 320000
D = 128
ALPHA = 0.1
K_PROP = 10

NC = 2
NS = 16
NW = NC * NS

NPAD = 10240
DEG_PER_W = NPAD // NS

CHUNK = 100
NCHUNKS = (E // NW) // CHUNK

DCHUNK = 80
DNCHUNKS = (E // NS) // DCHUNK

ACC_ROWS = NPAD
ROWS_PER_W = ACC_ROWS // NS
ZROWS = 64

_mesh = plsc.VectorSubcoreMesh(core_axis_name="c", subcore_axis_name="s")


def _fill(ref, nwords, value):
    v = jnp.full((16,), value, ref.dtype)
    for i in range(nwords // 16):
        ref[pl.ds(i * 16, 16)] = v


@functools.partial(
    pl.kernel,
    out_type=jax.ShapeDtypeStruct((NC, NPAD), jnp.float32),
    mesh=_mesh,
    scratch_types=[
        pltpu.VMEM((DNCHUNKS, DCHUNK), jnp.int32),
        pltpu.VMEM((DCHUNK,), jnp.float32),
        pltpu.VMEM((DEG_PER_W,), jnp.float32),
        pltpu.VMEM_SHARED((NPAD,), jnp.float32),
    ],
)
def _deg_kernel(ed_hbm, out_hbm, idx_v, ones_v, zeros_v, acc_sh):
    c = lax.axis_index("c")
    s = lax.axis_index("s")
    _fill(ones_v, DCHUNK, 1.0)
    _fill(zeros_v, DEG_PER_W, 0.0)
    pltpu.sync_copy(zeros_v, acc_sh.at[pl.ds(s * DEG_PER_W, DEG_PER_W)])
    plsc.subcore_barrier()
    pltpu.sync_copy(ed_hbm.at[c, s], idx_v)

    def chunk(j, carry):
        pltpu.sync_copy(ones_v, acc_sh.at[idx_v.at[j]], add=True)
        return carry

    lax.fori_loop(0, DNCHUNKS, chunk, 0, unroll=False)
    plsc.subcore_barrier()
    sl = pl.ds(s * DEG_PER_W, DEG_PER_W)
    pltpu.sync_copy(acc_sh.at[sl], out_hbm.at[c, sl])


@functools.partial(
    pl.kernel,
    out_type=jax.ShapeDtypeStruct((NC, ACC_ROWS, D), jnp.float32),
    mesh=_mesh,
    scratch_types=[
        pltpu.VMEM((NCHUNKS, CHUNK), jnp.int32),
        pltpu.VMEM((NCHUNKS, CHUNK), jnp.int32),
        pltpu.VMEM((CHUNK, D), jnp.float32),
        pltpu.VMEM((ZROWS, D), jnp.float32),
        pltpu.VMEM_SHARED((ACC_ROWS, D), jnp.float32),
        pltpu.SemaphoreType.DMA,
    ],
)
def _spmm_kernel(g_hbm, src_hbm, dst_hbm, out_hbm,
                 src_v, dst_v, rows_v, zbuf_v, acc_sh, gsem):
    c = lax.axis_index("c")
    s = lax.axis_index("s")
    wid = s * NC + c

    def zrow(i, carry):
        for l in range(D // 16):
            zbuf_v[i, pl.ds(l * 16, 16)] = jnp.zeros((16,), jnp.float32)
        return carry

    lax.fori_loop(0, ZROWS, zrow, 0, unroll=False)
    base = s * ROWS_PER_W
    for i in range(ROWS_PER_W // ZROWS):
        pltpu.sync_copy(zbuf_v, acc_sh.at[pl.ds(base + i * ZROWS, ZROWS)])
    plsc.subcore_barrier()

    pltpu.sync_copy(src_hbm.at[wid], src_v)
    pltpu.sync_copy(dst_hbm.at[wid], dst_v)

    def chunk(j, carry):
        pltpu.async_copy(g_hbm.at[src_v.at[j]], rows_v, gsem).wait()
        pltpu.sync_copy(rows_v, acc_sh.at[dst_v.at[j]], add=True)
        return carry

    lax.fori_loop(0, NCHUNKS, chunk, 0, unroll=False)
    plsc.subcore_barrier()

    for i in range(ROWS_PER_W // ZROWS):
        sl = pl.ds(base + i * ZROWS, ZROWS)
        pltpu.sync_copy(acc_sh.at[sl], out_hbm.at[c, sl])


def _norm_body(deg_ref, out_ref):
    d = deg_ref[...]
    out_ref[...] = jnp.where(d > 0, lax.rsqrt(jnp.maximum(d, 1e-12)), 0.0)


def _norms_call(deg):
    return pl.pallas_call(
        _norm_body,
        out_shape=jax.ShapeDtypeStruct((NC, NPAD), jnp.float32),
    )(deg)


def _mm_body(x_ref, w_ref, ns_ref, o_ref):
    xw = jnp.dot(x_ref[...], w_ref[...], preferred_element_type=jnp.float32)
    o_ref[...] = xw * ns_ref[...]


def _mm_call(x, w, ns_col):
    grid = 10
    blk = N // grid
    return pl.pallas_call(
        _mm_body,
        grid=(grid,),
        in_specs=[
            pl.BlockSpec((blk, D), lambda i: (i, 0)),
            pl.BlockSpec((D, D), lambda i: (0, 0)),
            pl.BlockSpec((blk, 1), lambda i: (i, 0)),
        ],
        out_specs=pl.BlockSpec((blk, D), lambda i: (i, 0)),
        out_shape=jax.ShapeDtypeStruct((N, D), jnp.float32),
    )(x, w, ns_col)


def _gcn_body(p_ref, b_ref, nd_ref, ns_ref, h_ref, g_ref):
    t = (p_ref[0] + p_ref[1]) * nd_ref[...]
    h = jnp.maximum(t + b_ref[...], 0.0)
    h_ref[...] = h
    g_ref[...] = h * ns_ref[...]


def _appnp_body(p_ref, h0_ref, nd_ref, ns_ref, h_ref, g_ref):
    t = (p_ref[0] + p_ref[1]) * nd_ref[...]
    h = (1.0 - ALPHA) * t + ALPHA * h0_ref[...]
    h_ref[...] = h
    g_ref[...] = h * ns_ref[...]


def _combine_call(body, p, extra, extra_is_full, nd_col, ns_col):
    grid = 10
    blk = N // grid
    col = pl.BlockSpec((blk, 1), lambda i: (i, 0))
    mat = pl.BlockSpec((blk, D), lambda i: (i, 0))
    extra_spec = mat if extra_is_full else pl.BlockSpec((1, D), lambda i: (0, 0))
    return pl.pallas_call(
        body,
        grid=(grid,),
        in_specs=[
            pl.BlockSpec((NC, blk, D), lambda i: (0, i, 0)),
            extra_spec, col, col,
        ],
        out_specs=[mat, mat],
        out_shape=[jax.ShapeDtypeStruct((N, D), jnp.float32)] * 2,
    )(p, extra, nd_col, ns_col)


@jax.jit
def kernel(features, edge_index, W, b):
    src = edge_index[0].astype(jnp.int32)
    dst = edge_index[1].astype(jnp.int32)
    ed = jnp.stack([src, dst]).reshape(2, NS, DNCHUNKS, DCHUNK)
    srcr = src.reshape(NW, NCHUNKS, CHUNK)
    dstr = dst.reshape(NW, NCHUNKS, CHUNK)

    deg = _deg_kernel(ed)
    norms = _norms_call(deg)
    ns_col = norms[0, :N].reshape(N, 1)
    nd_col = norms[1, :N].reshape(N, 1)

    g = _mm_call(features, W, ns_col)
    p = _spmm_kernel(g, srcr, dstr)
    h, g = _combine_call(_gcn_body, p, b.reshape(1, D), False, nd_col, ns_col)
    h0 = h
    for _ in range(K_PROP):
        p = _spmm_kernel(g, srcr, dstr)
        h, g = _combine_call(_appnp_body, p, h0, True, nd_col, ns_col)
    return h
